# baseline (speedup 1.0000x reference)
import jax
import jax.numpy as jnp
from jax.experimental import pallas as pl
from jax.experimental.pallas import tpu as pltpu

B, T, N = 128, 1024, 3
HID = 64
G4 = 4 * HID


def _lstm_kernel(xT_ref, wgcn_ref, bgcn_ref, wih_ref, whh_ref, bvec_ref,
                 wfc_ref, bfc_ref, out_ref, xm_ref):
    xm_ref[:] = (xT_ref[0] + xT_ref[1] + xT_ref[2]) * (1.0 / 3.0)

    v = jnp.dot(wih_ref[:], wgcn_ref[:], preferred_element_type=jnp.float32)
    u = jnp.dot(wih_ref[:], bgcn_ref[:],
                preferred_element_type=jnp.float32) + bvec_ref[:]
    waug = jnp.concatenate([whh_ref[:], v, u], axis=1)
    ones_row = jnp.ones((1, B), jnp.float32)

    def stanh(z):
        return 2.0 * jax.nn.sigmoid(2.0 * z) - 1.0

    def step(t, carry):
        h, c = carry
        xrow = xm_ref[pl.ds(t, 1), :]
        haug = jnp.concatenate([h, xrow, ones_row], axis=0)
        gates = jnp.dot(waug, haug, preferred_element_type=jnp.float32)
        i = jax.nn.sigmoid(gates[0 * HID:1 * HID])
        f = jax.nn.sigmoid(gates[1 * HID:2 * HID])
        g = stanh(gates[2 * HID:3 * HID])
        o = jax.nn.sigmoid(gates[3 * HID:4 * HID])
        c2 = f * c + i * g
        h2 = o * stanh(c2)
        return (h2, c2)

    h0 = jnp.zeros((HID, B), jnp.float32)
    c0 = jnp.zeros((HID, B), jnp.float32)
    h, _ = jax.lax.fori_loop(0, T, step, (h0, c0))

    out_ref[:] = jnp.dot(wfc_ref[:], h,
                         preferred_element_type=jnp.float32) + bfc_ref[:]


def kernel(x, W_gcn, b_gcn, W_ih, W_hh, b_ih, b_hh, W_fc, b_fc):
    out_dim = W_fc.shape[0]
    xT = jnp.transpose(x, (2, 1, 0))
    wgcn_c = jnp.transpose(W_gcn)
    bgcn_c = b_gcn[:, None]
    bvec_c = (b_ih + b_hh)[:, None]
    wfc_p = jnp.pad(W_fc, ((0, 8 - out_dim), (0, 0)))
    bfc_p = jnp.pad(b_fc, (0, 8 - out_dim))[:, None]

    res = pl.pallas_call(
        _lstm_kernel,
        out_shape=jax.ShapeDtypeStruct((8, B), jnp.float32),
        scratch_shapes=[pltpu.VMEM((T, B), jnp.float32)],
    )(xT, wgcn_c, bgcn_c, W_ih, W_hh, bvec_c, wfc_p, bfc_p)

    return jnp.transpose(res[:out_dim, :])


# speedup vs baseline: 1.0622x; 1.0622x over previous
import jax
import jax.numpy as jnp
from jax.experimental import pallas as pl
from jax.experimental.pallas import tpu as pltpu

B, T, N = 128, 1024, 3
HID = 64
G4 = 4 * HID


def _lstm_kernel(xT_ref, wgcn_ref, bgcn_ref, wih_ref, whh_ref, bvec_ref,
                 wfc_ref, bfc_ref, out_ref, xm_ref):
    xm_ref[:] = (xT_ref[0] + xT_ref[1] + xT_ref[2]) * (1.0 / 3.0)

    v = jnp.dot(wih_ref[:], wgcn_ref[:], preferred_element_type=jnp.float32)
    u = jnp.dot(wih_ref[:], bgcn_ref[:],
                preferred_element_type=jnp.float32) + bvec_ref[:]
    waug = jnp.concatenate([whh_ref[:], v, u], axis=1)
    ones_row = jnp.ones((1, B), jnp.float32)

    def step(t, carry):
        h, c, xrow = carry
        haug = jnp.concatenate([h, xrow, ones_row], axis=0)
        gates = jnp.dot(waug, haug, preferred_element_type=jnp.float32)
        xnext = xm_ref[pl.ds(jnp.minimum(t + 1, T - 1), 1), :]
        i = jax.nn.sigmoid(gates[0 * HID:1 * HID])
        f = jax.nn.sigmoid(gates[1 * HID:2 * HID])
        g = jnp.tanh(gates[2 * HID:3 * HID])
        o = jax.nn.sigmoid(gates[3 * HID:4 * HID])
        c2 = f * c + i * g
        h2 = o * jnp.tanh(c2)
        return (h2, c2, xnext)

    h0 = jnp.zeros((HID, B), jnp.float32)
    c0 = jnp.zeros((HID, B), jnp.float32)
    x0 = xm_ref[pl.ds(0, 1), :]
    h, _, _ = jax.lax.fori_loop(0, T, step, (h0, c0, x0), unroll=2)

    out_ref[:] = jnp.dot(wfc_ref[:], h,
                         preferred_element_type=jnp.float32) + bfc_ref[:]


def kernel(x, W_gcn, b_gcn, W_ih, W_hh, b_ih, b_hh, W_fc, b_fc):
    out_dim = W_fc.shape[0]
    xT = jnp.transpose(x, (2, 1, 0))
    wgcn_c = jnp.transpose(W_gcn)
    bgcn_c = b_gcn[:, None]
    bvec_c = (b_ih + b_hh)[:, None]
    wfc_p = jnp.pad(W_fc, ((0, 8 - out_dim), (0, 0)))
    bfc_p = jnp.pad(b_fc, (0, 8 - out_dim))[:, None]

    res = pl.pallas_call(
        _lstm_kernel,
        out_shape=jax.ShapeDtypeStruct((8, B), jnp.float32),
        scratch_shapes=[pltpu.VMEM((T, B), jnp.float32)],
    )(xT, wgcn_c, bgcn_c, W_ih, W_hh, bvec_c, wfc_p, bfc_p)

    return jnp.transpose(res[:out_dim, :])


# speedup vs baseline: 1.0622x; 1.0001x over previous
import jax
import jax.numpy as jnp
from jax.experimental import pallas as pl
from jax.experimental.pallas import tpu as pltpu

B, T, N = 128, 1024, 3
HID = 64
G4 = 4 * HID


def _lstm_kernel(xT_ref, wgcn_ref, bgcn_ref, wih_ref, whh_ref, bvec_ref,
                 wfc_ref, bfc_ref, out_ref, xm_ref):
    xm_ref[:] = (xT_ref[0] + xT_ref[1] + xT_ref[2]) * (1.0 / 3.0)

    v = jnp.dot(wih_ref[:], wgcn_ref[:], preferred_element_type=jnp.float32)
    u = jnp.dot(wih_ref[:], bgcn_ref[:],
                preferred_element_type=jnp.float32) + bvec_ref[:]
    whh = whh_ref[:]

    def step(t, carry):
        h, c, xrow = carry
        gates = jnp.dot(whh, h, preferred_element_type=jnp.float32) + (v * xrow + u)
        xnext = xm_ref[pl.ds(jnp.minimum(t + 1, T - 1), 1), :]
        i = jax.nn.sigmoid(gates[0 * HID:1 * HID])
        f = jax.nn.sigmoid(gates[1 * HID:2 * HID])
        g = jnp.tanh(gates[2 * HID:3 * HID])
        o = jax.nn.sigmoid(gates[3 * HID:4 * HID])
        c2 = f * c + i * g
        h2 = o * jnp.tanh(c2)
        return (h2, c2, xnext)

    h0 = jnp.zeros((HID, B), jnp.float32)
    c0 = jnp.zeros((HID, B), jnp.float32)
    x0 = xm_ref[pl.ds(0, 1), :]
    h, _, _ = jax.lax.fori_loop(0, T, step, (h0, c0, x0), unroll=2)

    out_ref[:] = jnp.dot(wfc_ref[:], h,
                         preferred_element_type=jnp.float32) + bfc_ref[:]


def kernel(x, W_gcn, b_gcn, W_ih, W_hh, b_ih, b_hh, W_fc, b_fc):
    out_dim = W_fc.shape[0]
    xT = jnp.transpose(x, (2, 1, 0))
    wgcn_c = jnp.transpose(W_gcn)
    bgcn_c = b_gcn[:, None]
    bvec_c = (b_ih + b_hh)[:, None]
    wfc_p = jnp.pad(W_fc, ((0, 8 - out_dim), (0, 0)))
    bfc_p = jnp.pad(b_fc, (0, 8 - out_dim))[:, None]

    res = pl.pallas_call(
        _lstm_kernel,
        out_shape=jax.ShapeDtypeStruct((8, B), jnp.float32),
        scratch_shapes=[pltpu.VMEM((T, B), jnp.float32)],
    )(xT, wgcn_c, bgcn_c, W_ih, W_hh, bvec_c, wfc_p, bfc_p)

    return jnp.transpose(res[:out_dim, :])
